# baseline (speedup 1.0000x reference)
import jax
import jax.numpy as jnp
from jax import lax
from jax.experimental import pallas as pl
from jax.experimental.pallas import tpu as pltpu
from jax.experimental.pallas import tpu_sc as plsc

HIDDEN = 128
HALF = HIDDEN // 2
NUM_RELS = 64
NUM_TYPES = 16
DEG = 32
N_DST0 = 10000
N_DST1 = 10000
N_SRC0 = 20000
N_SRC1 = 10000
E0 = N_DST0 * DEG
E1 = N_DST1 * DEG
BETA = 0.3

NW = 32
CH_D0 = 8
CH_E = CH_D0 * DEG
NCHUNK0 = N_DST0 // CH_D0
KMAX0 = (NCHUNK0 + NW - 1) // NW
NCHUNK1 = E1 // CH_E
KMAX1 = (NCHUNK1 + NW - 1) // NW

TILE0 = 400
TILE1 = 80


def _unpack_f32(w):
    lo = lax.bitcast_convert_type(w << 16, jnp.float32)
    hi = lax.bitcast_convert_type(w & jnp.int32(-65536), jnp.float32)
    return jnp.concatenate([lo, hi], axis=-1)


def _pack_i32(x):
    b = lax.bitcast_convert_type(x, jnp.int32)
    r = (b + 0x7FFF + ((b >> 16) & 1)) >> 16
    lo = r[..., :HALF] & 0xFFFF
    hi = r[..., HALF:] << 16
    return lo | hi


def _sc1_body(entity, src_ids0, edge_src0, aggE,
              src_tab, echunk, idxbuf, rows, accbuf, sem):
    wid = lax.axis_index("s") * 2 + lax.axis_index("c")
    pltpu.sync_copy(src_ids0, src_tab)

    def chunk_body(k, carry):
        c = wid + k * NW

        @pl.when(c < NCHUNK0)
        def _():
            pltpu.sync_copy(edge_src0.at[pl.ds(c * CH_E, CH_E)], echunk)
            for j in range(CH_E // 16):
                ev = echunk[pl.ds(j * 16, 16)]
                idxbuf[pl.ds(j * 16, 16)] = plsc.load_gather(src_tab, [ev])
            pltpu.async_copy(entity.at[idxbuf], rows, sem).wait()
            for d in range(CH_D0):
                def acc_body(kk, accs):
                    return tuple(
                        accs[j] + plsc.bitcast(
                            rows[d * DEG + kk, pl.ds(j * 16, 16)],
                            jnp.bfloat16)
                        for j in range(HALF // 16))
                accs = lax.fori_loop(
                    0, DEG, acc_body,
                    tuple(jnp.zeros((32,), jnp.bfloat16)
                          for _ in range(HALF // 16)),
                    unroll=4)
                for j in range(HALF // 16):
                    accbuf[d, pl.ds(j * 16, 16)] = plsc.bitcast(
                        accs[j], jnp.int32)
            pltpu.sync_copy(accbuf, aggE.at[pl.ds(c * CH_D0, CH_D0)])
        return carry

    lax.fori_loop(0, KMAX0, chunk_body, 0)


def _sc1(entity_p, src_ids0, edge_src0):
    mesh = plsc.VectorSubcoreMesh(core_axis_name="c", subcore_axis_name="s")
    f = pl.kernel(
        _sc1_body,
        out_type=jax.ShapeDtypeStruct((N_DST0, HALF), jnp.int32),
        mesh=mesh,
        scratch_types=[
            pltpu.VMEM((N_SRC0,), jnp.int32),
            pltpu.VMEM((CH_E,), jnp.int32),
            pltpu.VMEM((CH_E,), jnp.int32),
            pltpu.VMEM((CH_E, HALF), jnp.int32),
            pltpu.VMEM((CH_D0, HALF), jnp.int32),
            pltpu.SemaphoreType.DMA,
        ],
        compiler_params=pltpu.CompilerParams(needs_layout_passes=False, use_tc_tiling_on_sc=False),
    )
    return f(entity_p, src_ids0, edge_src0)


def _sc2_body(entity, src_ids1, edge_src1, agg, msg_g, aggm_g,
              src_tab, echunk, idxbuf, rows1, rows2, sem1, sem2):
    wid = lax.axis_index("s") * 2 + lax.axis_index("c")
    pltpu.sync_copy(src_ids1, src_tab)

    def chunk_body(k, carry):
        c = wid + k * NW

        @pl.when(c < NCHUNK1)
        def _():
            base = c * CH_E
            pltpu.sync_copy(edge_src1.at[pl.ds(base, CH_E)], echunk)
            for j in range(CH_E // 16):
                ev = echunk[pl.ds(j * 16, 16)]
                idxbuf[pl.ds(j * 16, 16)] = plsc.load_gather(src_tab, [ev])
            cp1 = pltpu.async_copy(entity.at[idxbuf], rows1, sem1)
            cp2 = pltpu.async_copy(agg.at[echunk], rows2, sem2)
            cp1.wait()
            cp2.wait()
            pltpu.sync_copy(rows1, msg_g.at[pl.ds(base, CH_E)])
            pltpu.sync_copy(rows2, aggm_g.at[pl.ds(base, CH_E)])
        return carry

    lax.fori_loop(0, KMAX1, chunk_body, 0)


def _sc2(entity_p, src_ids1, edge_src1, agg_p):
    mesh = plsc.VectorSubcoreMesh(core_axis_name="c", subcore_axis_name="s")
    f = pl.kernel(
        _sc2_body,
        out_type=(
            jax.ShapeDtypeStruct((E1, HALF), jnp.int32),
            jax.ShapeDtypeStruct((E1, HALF), jnp.int32),
        ),
        mesh=mesh,
        scratch_types=[
            pltpu.VMEM((N_SRC1,), jnp.int32),
            pltpu.VMEM((CH_E,), jnp.int32),
            pltpu.VMEM((CH_E,), jnp.int32),
            pltpu.VMEM((CH_E, HALF), jnp.int32),
            pltpu.VMEM((CH_E, HALF), jnp.int32),
            pltpu.SemaphoreType.DMA,
            pltpu.SemaphoreType.DMA,
        ],
        compiler_params=pltpu.CompilerParams(needs_layout_passes=False, use_tc_tiling_on_sc=False),
    )
    return f(entity_p, src_ids1, edge_src1, agg_p)


def _signed_onehot(et, n):
    r = et % NUM_RELS
    sign = jnp.where(et >= NUM_RELS, -1.0, 1.0).astype(jnp.float32)
    oneh = (lax.broadcasted_iota(jnp.int32, (n, NUM_RELS), 1) == r[:, None])
    return oneh.astype(jnp.float32) * sign[:, None]


def _tc1_body(aggE_ref, et_ref, rel_ref, out_ref):
    ne = TILE0 * DEG
    et = et_ref[0, 0, :]
    oneh = _signed_onehot(et, ne)
    cnt = oneh.reshape(TILE0, DEG, NUM_RELS).sum(axis=1)
    aggR = jnp.dot(cnt, rel_ref[...], preferred_element_type=jnp.float32)
    agg = (_unpack_f32(aggE_ref[...]) + aggR) * (1.0 / DEG)
    out_ref[...] = _pack_i32(agg)


def _tc1(aggE_p, etype0_r, relation, interpret=False):
    grid = N_DST0 // TILE0
    return pl.pallas_call(
        _tc1_body,
        grid=(grid,),
        in_specs=[
            pl.BlockSpec((TILE0, HALF), lambda i: (i, 0)),
            pl.BlockSpec((1, 1, TILE0 * DEG), lambda i: (i, 0, 0)),
            pl.BlockSpec((NUM_RELS, HIDDEN), lambda i: (0, 0)),
        ],
        out_specs=pl.BlockSpec((TILE0, HALF), lambda i: (i, 0)),
        out_shape=jax.ShapeDtypeStruct((N_DST0, HALF), jnp.int32),
        interpret=interpret,
    )(aggE_p, etype0_r, relation)


def _attn_pool(x, w_ref, b_ref):
    w = w_ref[...]
    b = b_ref[0:1, :]
    m1 = jnp.dot(jax.nn.relu(x), w, preferred_element_type=jnp.float32) + b
    xm = x.reshape(TILE1, DEG, HIDDEN).mean(axis=1)
    m2 = jnp.dot(jax.nn.relu(xm), w, preferred_element_type=jnp.float32) + b
    s1 = m1.mean(axis=-1).reshape(TILE1, DEG)
    s2 = m2.mean(axis=-1)[:, None]
    mx = jnp.maximum(s1.max(axis=1), s2[:, 0])
    e1 = jnp.exp(s1 - mx[:, None])
    e2 = jnp.exp(s2 - mx[:, None])
    z = e1.sum(axis=1) + e2[:, 0]
    w1 = e1 / z[:, None]
    w2 = e2 / z[:, None]
    pooled = (w1[:, :, None] * m1.reshape(TILE1, DEG, NUM_TYPES)).sum(axis=1)
    return pooled + w2 * m2


def _tc2_body(msg_ref, aggm_ref, et_ref, rel_ref, w_ref, b_ref, out_ref):
    ne = TILE1 * DEG
    et = et_ref[0, 0, :]
    rel2 = jnp.dot(_signed_onehot(et, ne), rel_ref[...],
                   preferred_element_type=jnp.float32)
    p1 = _attn_pool(_unpack_f32(msg_ref[...]) + rel2, w_ref, b_ref)
    p2 = _attn_pool(_unpack_f32(aggm_ref[...]) + rel2, w_ref, b_ref)
    out_ref[...] = jax.nn.sigmoid(BETA * p1 + (1.0 - BETA) * p2)


def _tc2(msg_p, aggm_p, etype1_r, relation, fc_W, fc_b2, interpret=False):
    grid = N_DST1 // TILE1
    ne = TILE1 * DEG
    return pl.pallas_call(
        _tc2_body,
        grid=(grid,),
        in_specs=[
            pl.BlockSpec((ne, HALF), lambda i: (i, 0)),
            pl.BlockSpec((ne, HALF), lambda i: (i, 0)),
            pl.BlockSpec((1, 1, ne), lambda i: (i, 0, 0)),
            pl.BlockSpec((NUM_RELS, HIDDEN), lambda i: (0, 0)),
            pl.BlockSpec((HIDDEN, NUM_TYPES), lambda i: (0, 0)),
            pl.BlockSpec((8, NUM_TYPES), lambda i: (0, 0)),
        ],
        out_specs=pl.BlockSpec((TILE1, NUM_TYPES), lambda i: (i, 0)),
        out_shape=jax.ShapeDtypeStruct((N_DST1, NUM_TYPES), jnp.float32),
        interpret=interpret,
    )(msg_p, aggm_p, etype1_r, relation, fc_W, fc_b2)


def _pack_table(x):
    xb = x.astype(jnp.bfloat16)
    lo = lax.bitcast_convert_type(xb[:, :HALF], jnp.uint16).astype(jnp.int32)
    hi = lax.bitcast_convert_type(xb[:, HALF:], jnp.uint16).astype(jnp.int32)
    return lo | (hi << 16)


def kernel(entity, relation, fc_W, fc_b,
           src_ids0, edge_src0, etype0, src_ids1, edge_src1, etype1):
    entity_p = _pack_table(entity)
    aggE_p = _sc1(entity_p, src_ids0, edge_src0)
    agg_p = _tc1(aggE_p, etype0.reshape(N_DST0 // TILE0, 1, TILE0 * DEG),
                 relation)
    msg_p, aggm_p = _sc2(entity_p, src_ids1, edge_src1, agg_p)
    fc_b2 = jnp.broadcast_to(fc_b[None, :], (8, NUM_TYPES))
    out = _tc2(msg_p, aggm_p,
               etype1.reshape(N_DST1 // TILE1, 1, TILE1 * DEG),
               relation, fc_W, fc_b2)
    return out


# speedup vs baseline: 1.2719x; 1.2719x over previous
import jax
import jax.numpy as jnp
from jax import lax
from jax.experimental import pallas as pl
from jax.experimental.pallas import tpu as pltpu
from jax.experimental.pallas import tpu_sc as plsc

HIDDEN = 128
HALF = HIDDEN // 2
NUM_RELS = 64
NUM_TYPES = 16
DEG = 32
N_DST0 = 10000
N_DST1 = 10000
N_SRC0 = 20000
N_SRC1 = 10000
E0 = N_DST0 * DEG
E1 = N_DST1 * DEG
BETA = 0.3

NW = 32
CH_D0 = 8
CH_E = CH_D0 * DEG
NCHUNK0 = N_DST0 // CH_D0
KMAX0 = (NCHUNK0 + NW - 1) // NW
NCHUNK1 = E1 // CH_E
KMAX1 = (NCHUNK1 + NW - 1) // NW

TILE0 = 400
TILE1 = 80


def _unpack_f32(w):
    lo = lax.bitcast_convert_type(w << 16, jnp.float32)
    hi = lax.bitcast_convert_type(w & jnp.int32(-65536), jnp.float32)
    return jnp.concatenate([lo, hi], axis=-1)


def _pack_i32(x):
    b = lax.bitcast_convert_type(x, jnp.int32)
    r = (b + 0x7FFF + ((b >> 16) & 1)) >> 16
    lo = r[..., :HALF] & 0xFFFF
    hi = r[..., HALF:] << 16
    return lo | hi


def _sc1_body(entity, src_ids0, edge_src0, src_ids1, aggE, src2,
              src_tab, echunk, idxbuf, rows, accbuf, sbuf, srows, sem, sem2):
    wid = lax.axis_index("s") * 2 + lax.axis_index("c")
    pltpu.sync_copy(src_ids0, src_tab)

    def chunk_body(k, carry):
        c = wid + k * NW

        @pl.when(c < NCHUNK0)
        def _():
            pltpu.sync_copy(edge_src0.at[pl.ds(c * CH_E, CH_E)], echunk)
            pltpu.sync_copy(src_ids1.at[pl.ds(c * CH_D0, CH_D0)], sbuf)
            for j in range(CH_E // 16):
                ev = echunk[pl.ds(j * 16, 16)]
                idxbuf[pl.ds(j * 16, 16)] = plsc.load_gather(src_tab, [ev])
            cp1 = pltpu.async_copy(entity.at[idxbuf], rows, sem)
            cp2 = pltpu.async_copy(entity.at[sbuf], srows, sem2)
            cp1.wait()
            for d in range(CH_D0):
                def acc_body(kk, accs):
                    return tuple(
                        accs[j] + rows[d * DEG + kk, pl.ds(j * 16, 16)]
                        for j in range(HIDDEN // 16))
                accs = lax.fori_loop(
                    0, DEG, acc_body,
                    tuple(jnp.zeros((16,), jnp.float32)
                          for _ in range(HIDDEN // 16)),
                    unroll=4)
                for j in range(HIDDEN // 16):
                    accbuf[d, pl.ds(j * 16, 16)] = accs[j]
            cp2.wait()
            pltpu.sync_copy(accbuf, aggE.at[pl.ds(c * CH_D0, CH_D0)])
            pltpu.sync_copy(srows, src2.at[pl.ds(c * CH_D0, CH_D0)])
        return carry

    lax.fori_loop(0, KMAX0, chunk_body, 0)


def _sc1(entity, src_ids0, edge_src0, src_ids1):
    mesh = plsc.VectorSubcoreMesh(core_axis_name="c", subcore_axis_name="s")
    f = pl.kernel(
        _sc1_body,
        out_type=(
            jax.ShapeDtypeStruct((N_DST0, HIDDEN), jnp.float32),
            jax.ShapeDtypeStruct((N_SRC1, HIDDEN), jnp.float32),
        ),
        mesh=mesh,
        scratch_types=[
            pltpu.VMEM((N_SRC0,), jnp.int32),
            pltpu.VMEM((CH_E,), jnp.int32),
            pltpu.VMEM((CH_E,), jnp.int32),
            pltpu.VMEM((CH_E, HIDDEN), jnp.float32),
            pltpu.VMEM((CH_D0, HIDDEN), jnp.float32),
            pltpu.VMEM((CH_D0,), jnp.int32),
            pltpu.VMEM((CH_D0, HIDDEN), jnp.float32),
            pltpu.SemaphoreType.DMA,
            pltpu.SemaphoreType.DMA,
        ],
        compiler_params=pltpu.CompilerParams(needs_layout_passes=False),
    )
    return f(entity, src_ids0, edge_src0, src_ids1)


def _sc2_body(table, edge_src1, out_g, echunk, rows, sem):
    wid = lax.axis_index("s") * 2 + lax.axis_index("c")

    def chunk_body(k, carry):
        c = wid + k * NW

        @pl.when(c < NCHUNK1)
        def _():
            base = c * CH_E
            pltpu.sync_copy(edge_src1.at[pl.ds(base, CH_E)], echunk)
            pltpu.async_copy(table.at[echunk], rows, sem).wait()
            pltpu.sync_copy(rows, out_g.at[pl.ds(base, CH_E)])
        return carry

    lax.fori_loop(0, KMAX1, chunk_body, 0)


def _sc2(table, edge_src1):
    mesh = plsc.VectorSubcoreMesh(core_axis_name="c", subcore_axis_name="s")
    f = pl.kernel(
        _sc2_body,
        out_type=jax.ShapeDtypeStruct((E1, HIDDEN), jnp.int32),
        mesh=mesh,
        scratch_types=[
            pltpu.VMEM((CH_E,), jnp.int32),
            pltpu.VMEM((CH_E, HIDDEN), jnp.int32),
            pltpu.SemaphoreType.DMA,
        ],
        compiler_params=pltpu.CompilerParams(needs_layout_passes=False),
    )
    return f(table, edge_src1)


def _signed_onehot(et, n):
    r = et % NUM_RELS
    sign = jnp.where(et >= NUM_RELS, -1.0, 1.0).astype(jnp.float32)
    oneh = (lax.broadcasted_iota(jnp.int32, (n, NUM_RELS), 1) == r[:, None])
    return oneh.astype(jnp.float32) * sign[:, None]


def _tc1_body(aggE_ref, src2_ref, et_ref, rel_ref, out_ref):
    ne = TILE0 * DEG
    et = et_ref[0, 0, :]
    oneh = _signed_onehot(et, ne)
    cnt = oneh.reshape(TILE0, DEG, NUM_RELS).sum(axis=1)
    aggR = jnp.dot(cnt, rel_ref[...], preferred_element_type=jnp.float32)
    agg = (aggE_ref[...] + aggR) * (1.0 / DEG)
    out_ref[...] = jnp.concatenate(
        [_pack_i32(src2_ref[...]), _pack_i32(agg)], axis=1)


def _tc1(aggE, src2, etype0_r, relation, interpret=False):
    grid = N_DST0 // TILE0
    return pl.pallas_call(
        _tc1_body,
        grid=(grid,),
        in_specs=[
            pl.BlockSpec((TILE0, HIDDEN), lambda i: (i, 0)),
            pl.BlockSpec((TILE0, HIDDEN), lambda i: (i, 0)),
            pl.BlockSpec((1, 1, TILE0 * DEG), lambda i: (i, 0, 0)),
            pl.BlockSpec((NUM_RELS, HIDDEN), lambda i: (0, 0)),
        ],
        out_specs=pl.BlockSpec((TILE0, HIDDEN), lambda i: (i, 0)),
        out_shape=jax.ShapeDtypeStruct((N_DST0, HIDDEN), jnp.int32),
        interpret=interpret,
    )(aggE, src2, etype0_r, relation)


def _attn_pool(x, w_ref, b_ref):
    w = w_ref[...]
    b = b_ref[0:1, :]
    m1 = jnp.dot(jax.nn.relu(x), w, preferred_element_type=jnp.float32) + b
    xm = x.reshape(TILE1, DEG, HIDDEN).mean(axis=1)
    m2 = jnp.dot(jax.nn.relu(xm), w, preferred_element_type=jnp.float32) + b
    s1 = m1.mean(axis=-1).reshape(TILE1, DEG)
    s2 = m2.mean(axis=-1)[:, None]
    mx = jnp.maximum(s1.max(axis=1), s2[:, 0])
    e1 = jnp.exp(s1 - mx[:, None])
    e2 = jnp.exp(s2 - mx[:, None])
    z = e1.sum(axis=1) + e2[:, 0]
    w1 = e1 / z[:, None]
    w2 = e2 / z[:, None]
    pooled = (w1[:, :, None] * m1.reshape(TILE1, DEG, NUM_TYPES)).sum(axis=1)
    return pooled + w2 * m2


def _tc2_body(g_ref, et_ref, rel_ref, w_ref, b_ref, out_ref):
    ne = TILE1 * DEG
    et = et_ref[0, 0, :]
    rel2 = jnp.dot(_signed_onehot(et, ne), rel_ref[...],
                   preferred_element_type=jnp.float32)
    g = g_ref[...]
    p1 = _attn_pool(_unpack_f32(g[:, :HALF]) + rel2, w_ref, b_ref)
    p2 = _attn_pool(_unpack_f32(g[:, HALF:]) + rel2, w_ref, b_ref)
    out_ref[...] = jax.nn.sigmoid(BETA * p1 + (1.0 - BETA) * p2)


def _tc2(g, etype1_r, relation, fc_W, fc_b2, interpret=False):
    grid = N_DST1 // TILE1
    ne = TILE1 * DEG
    return pl.pallas_call(
        _tc2_body,
        grid=(grid,),
        in_specs=[
            pl.BlockSpec((ne, HIDDEN), lambda i: (i, 0)),
            pl.BlockSpec((1, 1, ne), lambda i: (i, 0, 0)),
            pl.BlockSpec((NUM_RELS, HIDDEN), lambda i: (0, 0)),
            pl.BlockSpec((HIDDEN, NUM_TYPES), lambda i: (0, 0)),
            pl.BlockSpec((8, NUM_TYPES), lambda i: (0, 0)),
        ],
        out_specs=pl.BlockSpec((TILE1, NUM_TYPES), lambda i: (i, 0)),
        out_shape=jax.ShapeDtypeStruct((N_DST1, NUM_TYPES), jnp.float32),
        interpret=interpret,
    )(g, etype1_r, relation, fc_W, fc_b2)


def kernel(entity, relation, fc_W, fc_b,
           src_ids0, edge_src0, etype0, src_ids1, edge_src1, etype1):
    aggE, src2 = _sc1(entity, src_ids0, edge_src0, src_ids1)
    table = _tc1(aggE, src2,
                 etype0.reshape(N_DST0 // TILE0, 1, TILE0 * DEG), relation)
    g = _sc2(table, edge_src1)
    fc_b2 = jnp.broadcast_to(fc_b[None, :], (8, NUM_TYPES))
    out = _tc2(g, etype1.reshape(N_DST1 // TILE1, 1, TILE1 * DEG),
               relation, fc_W, fc_b2)
    return out


# speedup vs baseline: 1.5679x; 1.2327x over previous
import jax
import jax.numpy as jnp
from jax import lax
from jax.experimental import pallas as pl
from jax.experimental.pallas import tpu as pltpu
from jax.experimental.pallas import tpu_sc as plsc

HIDDEN = 128
HALF = HIDDEN // 2
NUM_RELS = 64
NUM_TYPES = 16
DEG = 32
N_DST0 = 10000
N_DST1 = 10000
N_SRC0 = 20000
N_SRC1 = 10000
E0 = N_DST0 * DEG
E1 = N_DST1 * DEG
BETA = 0.3

NW = 32
CH_D0 = 8
CH_E = CH_D0 * DEG
NCHUNK0 = N_DST0 // CH_D0
KMAX0 = (NCHUNK0 + NW - 1) // NW
NCHUNK1 = E1 // CH_E
KMAX1 = (NCHUNK1 + NW - 1) // NW

TILE0 = 400
TILE1 = 200


def _unpack_f32(w):
    lo = lax.bitcast_convert_type(w << 16, jnp.float32)
    hi = lax.bitcast_convert_type(w & jnp.int32(-65536), jnp.float32)
    return jnp.concatenate([lo, hi], axis=-1)


def _pack_i32(x):
    b = lax.bitcast_convert_type(x, jnp.int32)
    r = (b + 0x7FFF + ((b >> 16) & 1)) >> 16
    lo = r[..., :HALF] & 0xFFFF
    hi = r[..., HALF:] << 16
    return lo | hi


def _sc1_body(entity, src_ids0, edge_src0, src_ids1, aggE, src2,
              src_tab, echunk, idxbuf, rows, accbuf, sbuf, srows, sem, sem2):
    wid = lax.axis_index("s") * 2 + lax.axis_index("c")
    pltpu.sync_copy(src_ids0, src_tab)

    def chunk_body(k, carry):
        c = wid + k * NW

        @pl.when(c < NCHUNK0)
        def _():
            pltpu.sync_copy(edge_src0.at[pl.ds(c * CH_E, CH_E)], echunk)
            pltpu.sync_copy(src_ids1.at[pl.ds(c * CH_D0, CH_D0)], sbuf)
            for j in range(CH_E // 16):
                ev = echunk[pl.ds(j * 16, 16)]
                idxbuf[pl.ds(j * 16, 16)] = plsc.load_gather(src_tab, [ev])
            cp1 = pltpu.async_copy(entity.at[idxbuf], rows, sem)
            cp2 = pltpu.async_copy(entity.at[sbuf], srows, sem2)
            cp1.wait()
            for d in range(CH_D0):
                def acc_body(kk, accs):
                    return tuple(
                        accs[j] + rows[d * DEG + kk, pl.ds(j * 16, 16)]
                        for j in range(HIDDEN // 16))
                accs = lax.fori_loop(
                    0, DEG, acc_body,
                    tuple(jnp.zeros((16,), jnp.float32)
                          for _ in range(HIDDEN // 16)),
                    unroll=4)
                for j in range(HIDDEN // 16):
                    accbuf[d, pl.ds(j * 16, 16)] = accs[j]
            cp2.wait()
            pltpu.sync_copy(accbuf, aggE.at[pl.ds(c * CH_D0, CH_D0)])
            pltpu.sync_copy(srows, src2.at[pl.ds(c * CH_D0, CH_D0)])
        return carry

    lax.fori_loop(0, KMAX0, chunk_body, 0)


def _sc1(entity, src_ids0, edge_src0, src_ids1):
    mesh = plsc.VectorSubcoreMesh(core_axis_name="c", subcore_axis_name="s")
    f = pl.kernel(
        _sc1_body,
        out_type=(
            jax.ShapeDtypeStruct((N_DST0, HIDDEN), jnp.float32),
            jax.ShapeDtypeStruct((N_SRC1, HIDDEN), jnp.float32),
        ),
        mesh=mesh,
        scratch_types=[
            pltpu.VMEM((N_SRC0,), jnp.int32),
            pltpu.VMEM((CH_E,), jnp.int32),
            pltpu.VMEM((CH_E,), jnp.int32),
            pltpu.VMEM((CH_E, HIDDEN), jnp.float32),
            pltpu.VMEM((CH_D0, HIDDEN), jnp.float32),
            pltpu.VMEM((CH_D0,), jnp.int32),
            pltpu.VMEM((CH_D0, HIDDEN), jnp.float32),
            pltpu.SemaphoreType.DMA,
            pltpu.SemaphoreType.DMA,
        ],
        compiler_params=pltpu.CompilerParams(needs_layout_passes=False),
    )
    return f(entity, src_ids0, edge_src0, src_ids1)


def _sc2_body(table, edge_src1, out_g, echunk, rows, sem):
    wid = lax.axis_index("s") * 2 + lax.axis_index("c")

    def chunk_body(k, carry):
        c = wid + k * NW

        @pl.when(c < NCHUNK1)
        def _():
            base = c * CH_E
            pltpu.sync_copy(edge_src1.at[pl.ds(base, CH_E)], echunk)
            pltpu.async_copy(table.at[echunk], rows, sem).wait()
            pltpu.sync_copy(rows, out_g.at[pl.ds(base, CH_E)])
        return carry

    lax.fori_loop(0, KMAX1, chunk_body, 0)


def _sc2(table, edge_src1):
    mesh = plsc.VectorSubcoreMesh(core_axis_name="c", subcore_axis_name="s")
    f = pl.kernel(
        _sc2_body,
        out_type=jax.ShapeDtypeStruct((E1, HIDDEN), jnp.int32),
        mesh=mesh,
        scratch_types=[
            pltpu.VMEM((CH_E,), jnp.int32),
            pltpu.VMEM((CH_E, HIDDEN), jnp.int32),
            pltpu.SemaphoreType.DMA,
        ],
        compiler_params=pltpu.CompilerParams(needs_layout_passes=False),
    )
    return f(table, edge_src1)


def _signed_onehot(et, n):
    r = et % NUM_RELS
    sign = jnp.where(et >= NUM_RELS, -1.0, 1.0).astype(jnp.float32)
    oneh = (lax.broadcasted_iota(jnp.int32, (n, NUM_RELS), 1) == r[:, None])
    return oneh.astype(jnp.float32) * sign[:, None]


def _tc1_body(aggE_ref, src2_ref, et_ref, rel_ref, out_ref):
    ne = TILE0 * DEG
    et = et_ref[0, 0, :]
    oneh = _signed_onehot(et, ne)
    cnt = oneh.reshape(TILE0, DEG, NUM_RELS).sum(axis=1)
    aggR = jnp.dot(cnt, rel_ref[...], preferred_element_type=jnp.float32)
    agg = (aggE_ref[...] + aggR) * (1.0 / DEG)
    out_ref[...] = jnp.concatenate(
        [_pack_i32(src2_ref[...]), _pack_i32(agg)], axis=1)


def _tc1(aggE, src2, etype0_r, relation, interpret=False):
    grid = N_DST0 // TILE0
    return pl.pallas_call(
        _tc1_body,
        grid=(grid,),
        in_specs=[
            pl.BlockSpec((TILE0, HIDDEN), lambda i: (i, 0)),
            pl.BlockSpec((TILE0, HIDDEN), lambda i: (i, 0)),
            pl.BlockSpec((1, 1, TILE0 * DEG), lambda i: (i, 0, 0)),
            pl.BlockSpec((NUM_RELS, HIDDEN), lambda i: (0, 0)),
        ],
        out_specs=pl.BlockSpec((TILE0, HIDDEN), lambda i: (i, 0)),
        out_shape=jax.ShapeDtypeStruct((N_DST0, HIDDEN), jnp.int32),
        interpret=interpret,
    )(aggE, src2, etype0_r, relation)


def _attn_pool(x, w_ref, b_ref):
    w = w_ref[...]
    b = b_ref[0:1, :]
    m1 = jnp.dot(jax.nn.relu(x), w, preferred_element_type=jnp.float32) + b
    e1 = jnp.exp(m1[:, NUM_TYPES:NUM_TYPES + 1])
    wm = m1 * e1
    seg = wm.reshape(TILE1, DEG, HIDDEN).sum(axis=1)
    xm = x.reshape(TILE1, DEG, HIDDEN).sum(axis=1) * (1.0 / DEG)
    m2 = jnp.dot(jax.nn.relu(xm), w, preferred_element_type=jnp.float32) + b
    e2 = jnp.exp(m2[:, NUM_TYPES:NUM_TYPES + 1])
    num = seg[:, :NUM_TYPES] + m2[:, :NUM_TYPES] * e2
    den = seg[:, NUM_TYPES + 1:NUM_TYPES + 2] + e2
    return num / den


def _tc2_body(g_ref, et_ref, rel_ref, w_ref, b_ref, out_ref):
    ne = TILE1 * DEG
    et = et_ref[0, 0, :]
    rel2 = jnp.dot(_signed_onehot(et, ne), rel_ref[...],
                   preferred_element_type=jnp.float32)
    g = g_ref[...]
    p1 = _attn_pool(_unpack_f32(g[:, :HALF]) + rel2, w_ref, b_ref)
    p2 = _attn_pool(_unpack_f32(g[:, HALF:]) + rel2, w_ref, b_ref)
    out_ref[...] = jax.nn.sigmoid(BETA * p1 + (1.0 - BETA) * p2)


def _tc2(g, etype1_r, relation, fc_W, fc_b2, interpret=False):
    grid = N_DST1 // TILE1
    ne = TILE1 * DEG
    return pl.pallas_call(
        _tc2_body,
        grid=(grid,),
        in_specs=[
            pl.BlockSpec((ne, HIDDEN), lambda i: (i, 0)),
            pl.BlockSpec((1, 1, ne), lambda i: (i, 0, 0)),
            pl.BlockSpec((NUM_RELS, HIDDEN), lambda i: (0, 0)),
            pl.BlockSpec((HIDDEN, HIDDEN), lambda i: (0, 0)),
            pl.BlockSpec((8, HIDDEN), lambda i: (0, 0)),
        ],
        out_specs=pl.BlockSpec((TILE1, NUM_TYPES), lambda i: (i, 0)),
        out_shape=jax.ShapeDtypeStruct((N_DST1, NUM_TYPES), jnp.float32),
        interpret=interpret,
    )(g, etype1_r, relation, fc_W, fc_b2)


def kernel(entity, relation, fc_W, fc_b,
           src_ids0, edge_src0, etype0, src_ids1, edge_src1, etype1):
    aggE, src2 = _sc1(entity, src_ids0, edge_src0, src_ids1)
    table = _tc1(aggE, src2,
                 etype0.reshape(N_DST0 // TILE0, 1, TILE0 * DEG), relation)
    g = _sc2(table, edge_src1)
    wbar = fc_W.mean(axis=1, keepdims=True)
    w_aug = jnp.concatenate(
        [fc_W, wbar, jnp.zeros((HIDDEN, HIDDEN - NUM_TYPES - 1), fc_W.dtype)],
        axis=1)
    b_aug = jnp.concatenate(
        [fc_b, fc_b.mean(keepdims=True), jnp.ones((1,), fc_b.dtype),
         jnp.zeros((HIDDEN - NUM_TYPES - 2,), fc_b.dtype)])
    b_aug2 = jnp.broadcast_to(b_aug[None, :], (8, HIDDEN))
    out = _tc2(g, etype1.reshape(N_DST1 // TILE1, 1, TILE1 * DEG),
               relation, w_aug, b_aug2)
    return out
